# Initial kernel scaffold; baseline (speedup 1.0000x reference)
import jax
import jax.numpy as jnp
from jax.experimental import pallas as pl


def kernel(player_loc, player_conf, ball_conf, player_loc_t, player_conf_t, ball_conf_t):
    raise NotImplementedError("write your pallas kernel here")


# speedup vs baseline: 10.7408x; 10.7408x over previous
import functools

import jax
import jax.numpy as jnp
from jax.experimental import pallas as pl
from jax.experimental.pallas import tpu as pltpu

_NEG_POS_RATIO = 3


def _row_conf_sums(x0, x1, lab, num_priors):
    logz = jnp.logaddexp(x0, x1)
    loss = logz - x0
    ce_pos = logz - x1
    pos = lab > 0
    npos_i = jnp.sum(pos.astype(jnp.int32))
    npos_c = jnp.maximum(npos_i, 1)
    k = npos_c * _NEG_POS_RATIO
    cnt_neg = num_priors - npos_i

    pos_ce_sum = jnp.sum(jnp.where(pos, ce_pos, 0.0))
    neg_all_sum = jnp.sum(jnp.where(pos, 0.0, loss))

    def topk_neg_sum():
        bits = jax.lax.bitcast_convert_type(loss, jnp.uint32)
        sign = (bits >> jnp.uint32(31)).astype(jnp.uint32)
        flip = jnp.where(sign == jnp.uint32(1),
                         jnp.uint32(0xFFFFFFFF), jnp.uint32(0x80000000))
        ukey = bits ^ flip
        negm = jnp.logical_not(pos)

        def body(i, cur):
            bit = jnp.uint32(31) - i.astype(jnp.uint32)
            test = cur | (jnp.uint32(1) << bit)
            c = jnp.sum(jnp.where(negm & (ukey >= test), 1, 0))
            return jnp.where(c >= k, test, cur)

        t_key = jax.lax.fori_loop(0, 32, body, jnp.uint32(0))
        n_gt = jnp.sum(jnp.where(negm & (ukey > t_key), 1, 0))
        r = (k - n_gt).astype(jnp.float32)
        t_bits = jnp.where(t_key >= jnp.uint32(0x80000000),
                           t_key ^ jnp.uint32(0x80000000),
                           ~t_key)
        t_val = jax.lax.bitcast_convert_type(t_bits, jnp.float32)
        gt_sum = jnp.sum(jnp.where(negm & (ukey > t_key), loss, 0.0))
        return gt_sum + r * t_val

    neg_sum = jax.lax.cond(k >= cnt_neg, lambda: neg_all_sum, topk_neg_sum)
    return pos_ce_sum + neg_sum, npos_c.astype(jnp.float32)


def _ssd_kernel(pl_loc_ref, pl_loct_ref, pl_mask4_ref,
                pl_x0_ref, pl_x1_ref, pl_lab_ref,
                bl_x0_ref, bl_x1_ref, bl_lab_ref,
                out_l_ref, out_pc_ref, out_bc_ref,
                acc_ref, *, num_priors, num_rows):
    b = pl.program_id(0)

    @pl.when(b == 0)
    def _init():
        for i in range(5):
            acc_ref[i] = 0.0

    d = pl_loc_ref[0] - pl_loct_ref[0]
    a = jnp.abs(d)
    l1 = jnp.where(a < 1.0, 0.5 * a * a, a - 0.5)
    m4 = pl_mask4_ref[0].astype(jnp.float32)
    loc_sum = jnp.sum(l1 * m4)

    p_ce, p_np = _row_conf_sums(pl_x0_ref[0], pl_x1_ref[0], pl_lab_ref[0],
                                num_priors)
    b_ce, b_np = _row_conf_sums(bl_x0_ref[0], bl_x1_ref[0], bl_lab_ref[0],
                                num_priors)

    acc_ref[0] += loc_sum
    acc_ref[1] += p_ce
    acc_ref[2] += b_ce
    acc_ref[3] += p_np
    acc_ref[4] += b_np

    @pl.when(b == num_rows - 1)
    def _finish():
        np_p = acc_ref[3]
        out_l_ref[...] = jnp.broadcast_to(acc_ref[0] / np_p, (1, 1))
        out_pc_ref[...] = jnp.broadcast_to(acc_ref[1] / np_p, (1, 1))
        out_bc_ref[...] = jnp.broadcast_to(acc_ref[2] / acc_ref[4], (1, 1))


def kernel(player_loc, player_conf, ball_conf, player_loc_t, player_conf_t,
           ball_conf_t):
    B = player_loc.shape[0]
    player_loc = player_loc.reshape(B, -1, 4)
    P = player_loc.shape[1]

    pl_loc = player_loc.reshape(B, 1, 4 * P)
    pl_loct = player_loc_t.reshape(B, 1, 4 * P)
    pl_lab = player_conf_t.reshape(B, P).astype(jnp.int32)
    bl_lab = ball_conf_t.reshape(B, P).astype(jnp.int32)
    mask4 = jnp.broadcast_to((pl_lab > 0)[:, :, None].astype(jnp.int8),
                             (B, P, 4)).reshape(B, 1, 4 * P)

    pc = player_conf.reshape(B, P, 2)
    bc = ball_conf.reshape(B, P, 2)
    pl_x0 = pc[:, :, 0].reshape(B, 1, P)
    pl_x1 = pc[:, :, 1].reshape(B, 1, P)
    bl_x0 = bc[:, :, 0].reshape(B, 1, P)
    bl_x1 = bc[:, :, 1].reshape(B, 1, P)

    row4 = pl.BlockSpec((1, 1, 4 * P), lambda i: (i, 0, 0))
    row1 = pl.BlockSpec((1, 1, P), lambda i: (i, 0, 0))
    out_spec = pl.BlockSpec((1, 1), lambda i: (0, 0))
    out_ty = jax.ShapeDtypeStruct((1, 1), jnp.float32)

    out_l, out_pc, out_bc = pl.pallas_call(
        functools.partial(_ssd_kernel, num_priors=P, num_rows=B),
        grid=(B,),
        in_specs=[row4, row4, row4, row1, row1, row1, row1, row1, row1],
        out_specs=[out_spec, out_spec, out_spec],
        out_shape=[out_ty, out_ty, out_ty],
        scratch_shapes=[pltpu.SMEM((5,), jnp.float32)],
    )(pl_loc, pl_loct, mask4,
      pl_x0, pl_x1, pl_lab.reshape(B, 1, P),
      bl_x0, bl_x1, bl_lab.reshape(B, 1, P))

    return (out_l[0, 0], out_pc[0, 0], out_bc[0, 0])


# speedup vs baseline: 12.9122x; 1.2022x over previous
import functools

import jax
import jax.numpy as jnp
from jax.experimental import pallas as pl
from jax.experimental.pallas import tpu as pltpu

_NEG_POS_RATIO = 3
_ROWS_PER_STEP = 4
_SUBLANES = 8


def _row_conf_sums(s, lab, num_priors):
    ell = jnp.log1p(jnp.exp(-jnp.abs(s)))
    loss = jnp.maximum(s, 0.0) + ell
    ce_pos = jnp.maximum(-s, 0.0) + ell
    pos = lab > 0
    npos_i = jnp.sum(pos.astype(jnp.int32))
    npos_c = jnp.maximum(npos_i, 1)
    k = npos_c * _NEG_POS_RATIO
    cnt_neg = num_priors - npos_i

    pos_ce_sum = jnp.sum(jnp.where(pos, ce_pos, 0.0))
    neg_all_sum = jnp.sum(jnp.where(pos, 0.0, loss))

    def topk_neg_sum():
        bits = jax.lax.bitcast_convert_type(s, jnp.uint32)
        sign = (bits >> jnp.uint32(31)).astype(jnp.uint32)
        flip = jnp.where(sign == jnp.uint32(1),
                         jnp.uint32(0xFFFFFFFF), jnp.uint32(0x80000000))
        ukey = bits ^ flip
        negm = jnp.logical_not(pos)

        def body(i, cur):
            bit = jnp.uint32(31) - i.astype(jnp.uint32)
            test = cur | (jnp.uint32(1) << bit)
            c = jnp.sum(jnp.where(negm & (ukey >= test), 1, 0))
            return jnp.where(c >= k, test, cur)

        t_key = jax.lax.fori_loop(0, 32, body, jnp.uint32(0))
        n_gt = jnp.sum(jnp.where(negm & (ukey > t_key), 1, 0))
        r = (k - n_gt).astype(jnp.float32)
        t_bits = jnp.where(t_key >= jnp.uint32(0x80000000),
                           t_key ^ jnp.uint32(0x80000000),
                           ~t_key)
        t_val = jax.lax.bitcast_convert_type(t_bits, jnp.float32)
        t_loss = jnp.maximum(t_val, 0.0) + jnp.log1p(jnp.exp(-jnp.abs(t_val)))
        gt_sum = jnp.sum(jnp.where(negm & (ukey > t_key), loss, 0.0))
        return gt_sum + r * t_loss

    neg_sum = jax.lax.cond(k >= cnt_neg, lambda: neg_all_sum, topk_neg_sum)
    return pos_ce_sum + neg_sum, npos_c.astype(jnp.float32)


def _ssd_kernel(loc_ref, loct_ref, mask4_ref,
                s_p_ref, lab_p_ref, s_b_ref, lab_b_ref,
                out_l_ref, out_pc_ref, out_bc_ref,
                acc_ref, *, num_priors, num_steps):
    g = pl.program_id(0)

    @pl.when(g == 0)
    def _init():
        for i in range(5):
            acc_ref[i] = 0.0

    loc_sum = jnp.float32(0.0)
    p_ce = jnp.float32(0.0)
    b_ce = jnp.float32(0.0)
    p_np = jnp.float32(0.0)
    b_np = jnp.float32(0.0)
    for r in range(_ROWS_PER_STEP):
        d = loc_ref[r] - loct_ref[r]
        a = jnp.abs(d)
        l1 = jnp.where(a < 1.0, 0.5 * a * a, a - 0.5)
        loc_sum += jnp.sum(l1 * mask4_ref[r].astype(jnp.float32))

        ce, np_ = _row_conf_sums(s_p_ref[r], lab_p_ref[r], num_priors)
        p_ce += ce
        p_np += np_
        ce, np_ = _row_conf_sums(s_b_ref[r], lab_b_ref[r], num_priors)
        b_ce += ce
        b_np += np_

    acc_ref[0] += loc_sum
    acc_ref[1] += p_ce
    acc_ref[2] += b_ce
    acc_ref[3] += p_np
    acc_ref[4] += b_np

    @pl.when(g == num_steps - 1)
    def _finish():
        np_p = acc_ref[3]
        out_l_ref[...] = jnp.broadcast_to(acc_ref[0] / np_p, (1, 1))
        out_pc_ref[...] = jnp.broadcast_to(acc_ref[1] / np_p, (1, 1))
        out_bc_ref[...] = jnp.broadcast_to(acc_ref[2] / acc_ref[4], (1, 1))


def kernel(player_loc, player_conf, ball_conf, player_loc_t, player_conf_t,
           ball_conf_t):
    B = player_loc.shape[0]
    player_loc = player_loc.reshape(B, -1, 4)
    P = player_loc.shape[1]
    S = _SUBLANES
    W = P // S
    W4 = (4 * P) // S

    loc = player_loc.reshape(B, S, W4)
    loct = player_loc_t.reshape(B, S, W4)
    lab_p = player_conf_t.reshape(B, P).astype(jnp.int32)
    lab_b = ball_conf_t.reshape(B, P).astype(jnp.int32)
    mask4 = jnp.broadcast_to((lab_p > 0)[:, :, None].astype(jnp.int8),
                             (B, P, 4)).reshape(B, S, W4)

    pc = player_conf.reshape(B, P, 2)
    bc = ball_conf.reshape(B, P, 2)
    s_p = (pc[:, :, 1] - pc[:, :, 0]).reshape(B, S, W)
    s_b = (bc[:, :, 1] - bc[:, :, 0]).reshape(B, S, W)

    R = _ROWS_PER_STEP
    num_steps = B // R
    spec4 = pl.BlockSpec((R, S, W4), lambda i: (i, 0, 0))
    spec1 = pl.BlockSpec((R, S, W), lambda i: (i, 0, 0))
    out_spec = pl.BlockSpec((1, 1), lambda i: (0, 0))
    out_ty = jax.ShapeDtypeStruct((1, 1), jnp.float32)

    out_l, out_pc, out_bc = pl.pallas_call(
        functools.partial(_ssd_kernel, num_priors=P, num_steps=num_steps),
        grid=(num_steps,),
        in_specs=[spec4, spec4, spec4, spec1, spec1, spec1, spec1],
        out_specs=[out_spec, out_spec, out_spec],
        out_shape=[out_ty, out_ty, out_ty],
        scratch_shapes=[pltpu.SMEM((5,), jnp.float32)],
    )(loc, loct, mask4,
      s_p, lab_p.reshape(B, S, W), s_b, lab_b.reshape(B, S, W))

    return (out_l[0, 0], out_pc[0, 0], out_bc[0, 0])


# speedup vs baseline: 14.2833x; 1.1062x over previous
import functools

import jax
import jax.numpy as jnp
from jax.experimental import pallas as pl
from jax.experimental.pallas import tpu as pltpu

_NEG_POS_RATIO = 3
_ROWS = 8
_C_CONF = 1024
_C_LOC = 2048


def _softplus_pair(s):
    ell = jnp.log1p(jnp.exp(-jnp.abs(s)))
    return jnp.maximum(s, 0.0) + ell, jnp.maximum(-s, 0.0) + ell


def _ssd_kernel(loc_ref, loct_ref, mask4_ref,
                s_p_ref, lab_p_ref, s_b_ref, lab_b_ref,
                out_l_ref, out_pc_ref, out_bc_ref,
                accl_ref, accp_ref, accb_ref, sacc_ref,
                *, num_priors, num_steps):
    g = pl.program_id(0)

    @pl.when(g == 0)
    def _init():
        accl_ref[...] = jnp.zeros_like(accl_ref)
        accp_ref[...] = jnp.zeros_like(accp_ref)
        accb_ref[...] = jnp.zeros_like(accb_ref)
        for i in range(4):
            sacc_ref[i] = 0.0

    w4 = 4 * num_priors
    for c0 in range(0, w4, _C_LOC):
        cw = min(_C_LOC, w4 - c0)
        d = loc_ref[:, c0:c0 + cw] - loct_ref[:, c0:c0 + cw]
        a = jnp.abs(d)
        l1 = jnp.where(a < 1.0, 0.5 * a * a, a - 0.5)
        accl_ref[:, 0:cw] += l1 * mask4_ref[:, c0:c0 + cw]

    for s_ref, lab_ref, acc_ref, islot, cslot in (
            (s_p_ref, lab_p_ref, accp_ref, 0, 2),
            (s_b_ref, lab_b_ref, accb_ref, 1, 3)):
        npr = jnp.zeros((_ROWS, 1), jnp.float32)
        for c0 in range(0, num_priors, _C_CONF):
            cw = min(_C_CONF, num_priors - c0)
            s = s_ref[:, c0:c0 + cw]
            pos = lab_ref[:, c0:c0 + cw] > 0
            loss, ce_pos = _softplus_pair(s)
            acc_ref[:, 0:cw] += jnp.where(pos, ce_pos, loss)
            npr += jnp.sum(pos.astype(jnp.float32), axis=1, keepdims=True)

        npc = jnp.maximum(npr, 1.0)
        sacc_ref[islot] += jnp.sum(npc)
        kv = jnp.float32(_NEG_POS_RATIO) * npc
        cnt_neg = jnp.float32(num_priors) - npr
        rowflag = kv < cnt_neg

        @pl.when(jnp.any(rowflag))
        def _topk_correction(s_ref=s_ref, lab_ref=lab_ref,
                             rowflag=rowflag, kv=kv, cslot=cslot):
            s = s_ref[...]
            pos = lab_ref[...] > 0
            negm = jnp.logical_not(pos)
            loss, _ = _softplus_pair(s)
            bits = jax.lax.bitcast_convert_type(s, jnp.uint32)
            sign = bits >> jnp.uint32(31)
            flip = jnp.where(sign == jnp.uint32(1),
                             jnp.uint32(0xFFFFFFFF), jnp.uint32(0x80000000))
            ukey = bits ^ flip

            def body(i, cur):
                bit = jnp.uint32(31) - i.astype(jnp.uint32)
                test = cur | (jnp.uint32(1) << bit)
                cnt = jnp.sum(jnp.where(negm & (ukey >= test), 1.0, 0.0),
                              axis=1, keepdims=True)
                return jnp.where(cnt >= kv, test, cur)

            t_key = jax.lax.fori_loop(
                0, 32, body, jnp.zeros((_ROWS, 1), jnp.uint32))
            gtm = negm & (ukey > t_key)
            n_gt = jnp.sum(jnp.where(gtm, 1.0, 0.0), axis=1, keepdims=True)
            t_bits = jnp.where(t_key >= jnp.uint32(0x80000000),
                               t_key ^ jnp.uint32(0x80000000), ~t_key)
            t_val = jax.lax.bitcast_convert_type(t_bits, jnp.float32)
            t_loss, _ = _softplus_pair(t_val)
            gt_sum = jnp.sum(jnp.where(gtm, loss, 0.0), axis=1, keepdims=True)
            topk_row = gt_sum + (kv - n_gt) * t_loss
            neg_all_row = jnp.sum(jnp.where(pos, 0.0, loss),
                                  axis=1, keepdims=True)
            sacc_ref[cslot] += jnp.sum(
                jnp.where(rowflag, topk_row - neg_all_row, 0.0))

    @pl.when(g == num_steps - 1)
    def _finish():
        np_p = sacc_ref[0]
        l_tot = jnp.sum(accl_ref[...])
        pc_tot = jnp.sum(accp_ref[...]) + sacc_ref[2]
        bc_tot = jnp.sum(accb_ref[...]) + sacc_ref[3]
        out_l_ref[...] = jnp.broadcast_to(l_tot / np_p, (1, 1))
        out_pc_ref[...] = jnp.broadcast_to(pc_tot / np_p, (1, 1))
        out_bc_ref[...] = jnp.broadcast_to(bc_tot / sacc_ref[1], (1, 1))


def kernel(player_loc, player_conf, ball_conf, player_loc_t, player_conf_t,
           ball_conf_t):
    B = player_loc.shape[0]
    player_loc = player_loc.reshape(B, -1, 4)
    P = player_loc.shape[1]

    loc = player_loc.reshape(B, 4 * P)
    loct = player_loc_t.reshape(B, 4 * P)
    lab_p = player_conf_t.reshape(B, P).astype(jnp.int32)
    lab_b = ball_conf_t.reshape(B, P).astype(jnp.int32)
    mask4 = jnp.broadcast_to((lab_p > 0)[:, :, None].astype(jnp.float32),
                             (B, P, 4)).reshape(B, 4 * P)
    cvec = jnp.array([-1.0, 1.0], jnp.float32)
    s_p = jnp.sum(player_conf.reshape(B, P, 2) * cvec, axis=-1)
    s_b = jnp.sum(ball_conf.reshape(B, P, 2) * cvec, axis=-1)

    num_steps = B // _ROWS
    spec4 = pl.BlockSpec((_ROWS, 4 * P), lambda i: (i, 0))
    spec1 = pl.BlockSpec((_ROWS, P), lambda i: (i, 0))
    out_spec = pl.BlockSpec((1, 1), lambda i: (0, 0))
    out_ty = jax.ShapeDtypeStruct((1, 1), jnp.float32)

    out_l, out_pc, out_bc = pl.pallas_call(
        functools.partial(_ssd_kernel, num_priors=P, num_steps=num_steps),
        grid=(num_steps,),
        in_specs=[spec4, spec4, spec4, spec1, spec1, spec1, spec1],
        out_specs=[out_spec, out_spec, out_spec],
        out_shape=[out_ty, out_ty, out_ty],
        scratch_shapes=[pltpu.VMEM((_ROWS, _C_LOC), jnp.float32),
                        pltpu.VMEM((_ROWS, _C_CONF), jnp.float32),
                        pltpu.VMEM((_ROWS, _C_CONF), jnp.float32),
                        pltpu.SMEM((4,), jnp.float32)],
    )(loc, loct, mask4, s_p, lab_p, s_b, lab_b)

    return (out_l[0, 0], out_pc[0, 0], out_bc[0, 0])
